# baseline (speedup 1.0000x reference)
import functools

import jax
import jax.numpy as jnp
from jax import lax
from jax.experimental import pallas as pl
from jax.experimental.pallas import tpu as pltpu
from jax.experimental.pallas import tpu_sc as plsc

N = 10000
E = 320000
D = 128
H = 128

NC = 2
NS = 16
NW = NC * NS
CHUNK = 128
CH0 = 88
CH1 = 72
CHMAX = max(CH0, CH1)
CHT = NS * (CH0 + CH1)
EPAD = CHT * CHUNK
NPAD = 10240
RPT = NPAD // NS

_mesh = plsc.VectorSubcoreMesh(core_axis_name="c", subcore_axis_name="s")


UNROLL = 8


def _split_loop(cid, body):
    lo, hi = min(CH0, CH1), max(CH0, CH1)

    def group(base):
        def g(k, carry):
            for u in range(UNROLL):
                body(k * UNROLL + u, carry)
            return carry
        return g

    lax.fori_loop(0, lo // UNROLL, group(0), 0)
    if hi > lo:
        @pl.when(cid == (0 if CH0 > CH1 else 1))
        def _():
            lax.fori_loop(lo // UNROLL, hi // UNROLL, group(0), 0)


def _deg_body(dst3, zeros_z, ones128, degp, idx_v, ones_v, deg_sh):
    cid = lax.axis_index("c")
    sid = lax.axis_index("s")
    wid = cid * NS + sid
    pltpu.sync_copy(dst3.at[wid], idx_v)
    pltpu.sync_copy(ones128, ones_v)
    pltpu.sync_copy(zeros_z.at[pl.ds(sid * RPT, RPT)],
                    deg_sh.at[pl.ds(sid * RPT, RPT)])
    plsc.subcore_barrier()

    def body(j, carry):
        pltpu.sync_copy(ones_v, deg_sh.at[idx_v.at[j]], add=True)
        return carry

    _split_loop(cid, body)
    plsc.subcore_barrier()
    pltpu.sync_copy(deg_sh.at[pl.ds(sid * RPT, RPT)],
                    degp.at[cid, pl.ds(sid * RPT, RPT)])


_deg_call = pl.kernel(
    _deg_body,
    out_type=jax.ShapeDtypeStruct((NC, NPAD, D), jnp.float32),
    mesh=_mesh,
    scratch_types=[
        pltpu.VMEM((CHMAX, CHUNK), jnp.int32),
        pltpu.VMEM((CHUNK, D), jnp.float32),
        pltpu.VMEM_SHARED((NPAD, D), jnp.float32),
    ],
)


def _conv_body(y, src3, dst3, zeros_z, zp, src_v, dst_v, rows_v, sem, z_sh):
    cid = lax.axis_index("c")
    sid = lax.axis_index("s")
    wid = cid * NS + sid
    pltpu.sync_copy(src3.at[wid], src_v)
    pltpu.sync_copy(dst3.at[wid], dst_v)
    pltpu.sync_copy(zeros_z.at[pl.ds(sid * RPT, RPT)],
                    z_sh.at[pl.ds(sid * RPT, RPT)])
    plsc.subcore_barrier()

    def body(j, carry):
        pltpu.async_copy(y.at[src_v.at[j]], rows_v, sem).wait()
        pltpu.sync_copy(rows_v, z_sh.at[dst_v.at[j]], add=True)
        return carry

    _split_loop(cid, body)
    plsc.subcore_barrier()
    pltpu.sync_copy(z_sh.at[pl.ds(sid * RPT, RPT)],
                    zp.at[cid, pl.ds(sid * RPT, RPT)])


_conv_call = pl.kernel(
    _conv_body,
    out_type=jax.ShapeDtypeStruct((NC, NPAD, D), jnp.float32),
    mesh=_mesh,
    scratch_types=[
        pltpu.VMEM((CHMAX, CHUNK), jnp.int32),
        pltpu.VMEM((CHMAX, CHUNK), jnp.int32),
        pltpu.VMEM((CHUNK, D), jnp.float32),
        pltpu.SemaphoreType.DMA,
        pltpu.VMEM_SHARED((NPAD, D), jnp.float32),
    ],
)


BN = 2048
GRID = NPAD // BN


def _leaky(h):
    return jnp.where(h > 0, h, 0.01 * h)


def _tc1_body(x_ref, w_ref, degp_ref, y_ref, dinv_ref):
    deg = degp_ref[0, :, 0:1] + degp_ref[1, :, 0:1] + 1.0
    dinvb = jnp.broadcast_to(lax.rsqrt(deg), (BN, D))
    xw = jnp.dot(x_ref[...], w_ref[...], preferred_element_type=jnp.float32)
    y_ref[...] = xw * dinvb
    dinv_ref[...] = dinvb


_tc1_call = pl.pallas_call(
    _tc1_body,
    grid=(GRID,),
    in_specs=[
        pl.BlockSpec((BN, D), lambda i: (i, 0)),
        pl.BlockSpec((D, H), lambda i: (0, 0)),
        pl.BlockSpec((NC, BN, H), lambda i: (0, i, 0)),
    ],
    out_specs=[
        pl.BlockSpec((BN, H), lambda i: (i, 0)),
        pl.BlockSpec((BN, H), lambda i: (i, 0)),
    ],
    out_shape=[
        jax.ShapeDtypeStruct((NPAD, H), jnp.float32),
        jax.ShapeDtypeStruct((NPAD, H), jnp.float32),
    ],
)


def _tcmid_body(zp_ref, y_ref, dinv_ref, b_ref, w_ref, yn_ref):
    z = zp_ref[0] + zp_ref[1] + y_ref[...]
    h = _leaky(dinv_ref[...] * z + b_ref[...])
    yn_ref[...] = jnp.dot(h, w_ref[...],
                          preferred_element_type=jnp.float32) * dinv_ref[...]


_tcmid_call = pl.pallas_call(
    _tcmid_body,
    grid=(GRID,),
    in_specs=[
        pl.BlockSpec((NC, BN, H), lambda i: (0, i, 0)),
        pl.BlockSpec((BN, H), lambda i: (i, 0)),
        pl.BlockSpec((BN, H), lambda i: (i, 0)),
        pl.BlockSpec((1, H), lambda i: (0, 0)),
        pl.BlockSpec((H, H), lambda i: (0, 0)),
    ],
    out_specs=pl.BlockSpec((BN, H), lambda i: (i, 0)),
    out_shape=jax.ShapeDtypeStruct((NPAD, H), jnp.float32),
)


def _tcfin_body(zp_ref, y_ref, dinv_ref, b3_ref, f1w_ref, f1b_ref,
                f2w_ref, f2b_ref, f3w_ref, f3b_ref, out_ref):
    z = zp_ref[0] + zp_ref[1] + y_ref[...]
    h = _leaky(dinv_ref[...] * z + b3_ref[...])
    h = _leaky(jnp.dot(h, f1w_ref[...],
                       preferred_element_type=jnp.float32) + f1b_ref[...])
    h = _leaky(jnp.dot(h, f2w_ref[...],
                       preferred_element_type=jnp.float32) + f2b_ref[...])
    logits = jnp.dot(h, f3w_ref[...],
                     preferred_element_type=jnp.float32) + f3b_ref[...]
    m = jnp.max(logits, axis=1, keepdims=True)
    lse = jnp.log(jnp.sum(jnp.exp(logits - m), axis=1, keepdims=True)) + m
    out_ref[...] = logits - lse


_tcfin_call = pl.pallas_call(
    _tcfin_body,
    grid=(GRID,),
    in_specs=[
        pl.BlockSpec((NC, BN, H), lambda i: (0, i, 0)),
        pl.BlockSpec((BN, H), lambda i: (i, 0)),
        pl.BlockSpec((BN, H), lambda i: (i, 0)),
        pl.BlockSpec((1, H), lambda i: (0, 0)),
        pl.BlockSpec((H, H), lambda i: (0, 0)),
        pl.BlockSpec((1, H), lambda i: (0, 0)),
        pl.BlockSpec((H, H), lambda i: (0, 0)),
        pl.BlockSpec((1, H), lambda i: (0, 0)),
        pl.BlockSpec((H, 2), lambda i: (0, 0)),
        pl.BlockSpec((1, 2), lambda i: (0, 0)),
    ],
    out_specs=pl.BlockSpec((BN, 2), lambda i: (i, 0)),
    out_shape=jax.ShapeDtypeStruct((NPAD, 2), jnp.float32),
)


def kernel(x, edge_index, batch, conv1_w, conv1_b, conv2_w, conv2_b,
           conv3_w, conv3_b, fc1_w, fc1_b, fc2_w, fc2_b, fc3_w, fc3_b):
    del batch
    pad_idx = N + (jnp.arange(EPAD - E, dtype=jnp.int32) % (NPAD - N))

    def _worker_layout(e):
        flat = jnp.concatenate([e.astype(jnp.int32), pad_idx])
        cut = NS * CH0 * CHUNK
        e0 = flat[:cut].reshape(NS, CH0, CHUNK)
        e1 = flat[cut:].reshape(NS, CH1, CHUNK)
        e0 = jnp.pad(e0, ((0, 0), (0, CHMAX - CH0), (0, 0)),
                     constant_values=N)
        e1 = jnp.pad(e1, ((0, 0), (0, CHMAX - CH1), (0, 0)),
                     constant_values=N)
        return jnp.concatenate([e0, e1], axis=0)

    src = _worker_layout(edge_index[0])
    dst = _worker_layout(edge_index[1])
    xp = jnp.pad(x, ((0, NPAD - N), (0, 0)))
    zeros_z = jnp.zeros((NPAD, D), jnp.float32)
    ones128 = jnp.ones((CHUNK, D), jnp.float32)

    degp = _deg_call(dst, zeros_z, ones128)
    y1, dinv = _tc1_call(xp, conv1_w, degp)
    z1 = _conv_call(y1, src, dst, zeros_z)
    y2 = _tcmid_call(z1, y1, dinv, conv1_b.reshape(1, H), conv2_w)
    z2 = _conv_call(y2, src, dst, zeros_z)
    y3 = _tcmid_call(z2, y2, dinv, conv2_b.reshape(1, H), conv3_w)
    z3 = _conv_call(y3, src, dst, zeros_z)
    out = _tcfin_call(z3, y3, dinv, conv3_b.reshape(1, H),
                      fc1_w, fc1_b.reshape(1, H),
                      fc2_w, fc2_b.reshape(1, H),
                      fc3_w, fc3_b.reshape(1, 2))
    return out[:N]


# speedup vs baseline: 1.0727x; 1.0727x over previous
import functools

import jax
import jax.numpy as jnp
from jax import lax
from jax.experimental import pallas as pl
from jax.experimental.pallas import tpu as pltpu
from jax.experimental.pallas import tpu_sc as plsc

N = 10000
E = 320000
D = 128
H = 128

NC = 2
NS = 16
NW = NC * NS
CHUNK = 128
CH0 = 80
CH1 = 80
CHMAX = max(CH0, CH1)
CHT = NS * (CH0 + CH1)
EPAD = CHT * CHUNK
NPAD = 10240
RPT = NPAD // NS

_mesh = plsc.VectorSubcoreMesh(core_axis_name="c", subcore_axis_name="s")


UNROLL = 8


def _split_loop(cid, body):
    lo, hi = min(CH0, CH1), max(CH0, CH1)

    def group(base):
        def g(k, carry):
            for u in range(UNROLL):
                body(k * UNROLL + u, carry)
            return carry
        return g

    lax.fori_loop(0, lo // UNROLL, group(0), 0)
    if hi > lo:
        @pl.when(cid == (0 if CH0 > CH1 else 1))
        def _():
            lax.fori_loop(lo // UNROLL, hi // UNROLL, group(0), 0)


def _deg_body(dst3, zeros_z, ones128, degp, idx_v, ones_v, deg_sh):
    cid = lax.axis_index("c")
    sid = lax.axis_index("s")
    wid = cid * NS + sid
    pltpu.sync_copy(dst3.at[wid], idx_v)
    pltpu.sync_copy(ones128, ones_v)
    pltpu.sync_copy(zeros_z.at[pl.ds(sid * RPT, RPT)],
                    deg_sh.at[pl.ds(sid * RPT, RPT)])
    plsc.subcore_barrier()

    def body(j, carry):
        pltpu.sync_copy(ones_v, deg_sh.at[idx_v.at[j]], add=True)
        return carry

    _split_loop(cid, body)
    plsc.subcore_barrier()
    pltpu.sync_copy(deg_sh.at[pl.ds(sid * RPT, RPT)],
                    degp.at[cid, pl.ds(sid * RPT, RPT)])


_deg_call = pl.kernel(
    _deg_body,
    out_type=jax.ShapeDtypeStruct((NC, NPAD, D), jnp.float32),
    mesh=_mesh,
    scratch_types=[
        pltpu.VMEM((CHMAX, CHUNK), jnp.int32),
        pltpu.VMEM((CHUNK, D), jnp.float32),
        pltpu.VMEM_SHARED((NPAD, D), jnp.float32),
    ],
)


def _conv_body(y, src3, dst3, zeros_z, zp, src_v, dst_v, rows_v, sem, z_sh):
    cid = lax.axis_index("c")
    sid = lax.axis_index("s")
    wid = cid * NS + sid
    pltpu.sync_copy(src3.at[wid], src_v)
    pltpu.sync_copy(dst3.at[wid], dst_v)
    pltpu.sync_copy(zeros_z.at[pl.ds(sid * RPT, RPT)],
                    z_sh.at[pl.ds(sid * RPT, RPT)])
    plsc.subcore_barrier()

    def body(j, carry):
        pltpu.async_copy(y.at[src_v.at[j]], rows_v, sem).wait()
        pltpu.sync_copy(rows_v, z_sh.at[dst_v.at[j]], add=True)
        return carry

    _split_loop(cid, body)
    plsc.subcore_barrier()
    pltpu.sync_copy(z_sh.at[pl.ds(sid * RPT, RPT)],
                    zp.at[cid, pl.ds(sid * RPT, RPT)])


_conv_call = pl.kernel(
    _conv_body,
    out_type=jax.ShapeDtypeStruct((NC, NPAD, D), jnp.float32),
    mesh=_mesh,
    scratch_types=[
        pltpu.VMEM((CHMAX, CHUNK), jnp.int32),
        pltpu.VMEM((CHMAX, CHUNK), jnp.int32),
        pltpu.VMEM((CHUNK, D), jnp.float32),
        pltpu.SemaphoreType.DMA,
        pltpu.VMEM_SHARED((NPAD, D), jnp.float32),
    ],
)


BN = 2048
GRID = NPAD // BN


def _leaky(h):
    return jnp.where(h > 0, h, 0.01 * h)


def _tc1_body(x_ref, w_ref, degp_ref, y_ref, dinv_ref):
    deg = degp_ref[0, :, 0:1] + degp_ref[1, :, 0:1] + 1.0
    dinvb = jnp.broadcast_to(lax.rsqrt(deg), (BN, D))
    xw = jnp.dot(x_ref[...], w_ref[...], preferred_element_type=jnp.float32)
    y_ref[...] = xw * dinvb
    dinv_ref[...] = dinvb


_tc1_call = pl.pallas_call(
    _tc1_body,
    grid=(GRID,),
    in_specs=[
        pl.BlockSpec((BN, D), lambda i: (i, 0)),
        pl.BlockSpec((D, H), lambda i: (0, 0)),
        pl.BlockSpec((NC, BN, H), lambda i: (0, i, 0)),
    ],
    out_specs=[
        pl.BlockSpec((BN, H), lambda i: (i, 0)),
        pl.BlockSpec((BN, H), lambda i: (i, 0)),
    ],
    out_shape=[
        jax.ShapeDtypeStruct((NPAD, H), jnp.float32),
        jax.ShapeDtypeStruct((NPAD, H), jnp.float32),
    ],
)


def _tcmid_body(zp_ref, y_ref, dinv_ref, b_ref, w_ref, yn_ref):
    z = zp_ref[0] + zp_ref[1] + y_ref[...]
    h = _leaky(dinv_ref[...] * z + b_ref[...])
    yn_ref[...] = jnp.dot(h, w_ref[...],
                          preferred_element_type=jnp.float32) * dinv_ref[...]


_tcmid_call = pl.pallas_call(
    _tcmid_body,
    grid=(GRID,),
    in_specs=[
        pl.BlockSpec((NC, BN, H), lambda i: (0, i, 0)),
        pl.BlockSpec((BN, H), lambda i: (i, 0)),
        pl.BlockSpec((BN, H), lambda i: (i, 0)),
        pl.BlockSpec((1, H), lambda i: (0, 0)),
        pl.BlockSpec((H, H), lambda i: (0, 0)),
    ],
    out_specs=pl.BlockSpec((BN, H), lambda i: (i, 0)),
    out_shape=jax.ShapeDtypeStruct((NPAD, H), jnp.float32),
)


def _tcfin_body(zp_ref, y_ref, dinv_ref, b3_ref, f1w_ref, f1b_ref,
                f2w_ref, f2b_ref, f3w_ref, f3b_ref, out_ref):
    z = zp_ref[0] + zp_ref[1] + y_ref[...]
    h = _leaky(dinv_ref[...] * z + b3_ref[...])
    h = _leaky(jnp.dot(h, f1w_ref[...],
                       preferred_element_type=jnp.float32) + f1b_ref[...])
    h = _leaky(jnp.dot(h, f2w_ref[...],
                       preferred_element_type=jnp.float32) + f2b_ref[...])
    logits = jnp.dot(h, f3w_ref[...],
                     preferred_element_type=jnp.float32) + f3b_ref[...]
    m = jnp.max(logits, axis=1, keepdims=True)
    lse = jnp.log(jnp.sum(jnp.exp(logits - m), axis=1, keepdims=True)) + m
    out_ref[...] = logits - lse


_tcfin_call = pl.pallas_call(
    _tcfin_body,
    grid=(GRID,),
    in_specs=[
        pl.BlockSpec((NC, BN, H), lambda i: (0, i, 0)),
        pl.BlockSpec((BN, H), lambda i: (i, 0)),
        pl.BlockSpec((BN, H), lambda i: (i, 0)),
        pl.BlockSpec((1, H), lambda i: (0, 0)),
        pl.BlockSpec((H, H), lambda i: (0, 0)),
        pl.BlockSpec((1, H), lambda i: (0, 0)),
        pl.BlockSpec((H, H), lambda i: (0, 0)),
        pl.BlockSpec((1, H), lambda i: (0, 0)),
        pl.BlockSpec((H, 2), lambda i: (0, 0)),
        pl.BlockSpec((1, 2), lambda i: (0, 0)),
    ],
    out_specs=pl.BlockSpec((BN, 2), lambda i: (i, 0)),
    out_shape=jax.ShapeDtypeStruct((NPAD, 2), jnp.float32),
)


def kernel(x, edge_index, batch, conv1_w, conv1_b, conv2_w, conv2_b,
           conv3_w, conv3_b, fc1_w, fc1_b, fc2_w, fc2_b, fc3_w, fc3_b):
    del batch
    pad_idx = N + (jnp.arange(EPAD - E, dtype=jnp.int32) % (NPAD - N))

    def _worker_layout(e):
        flat = jnp.concatenate([e.astype(jnp.int32), pad_idx])
        cut = NS * CH0 * CHUNK
        e0 = flat[:cut].reshape(NS, CH0, CHUNK)
        e1 = flat[cut:].reshape(NS, CH1, CHUNK)
        e0 = jnp.pad(e0, ((0, 0), (0, CHMAX - CH0), (0, 0)),
                     constant_values=N)
        e1 = jnp.pad(e1, ((0, 0), (0, CHMAX - CH1), (0, 0)),
                     constant_values=N)
        return jnp.concatenate([e0, e1], axis=0)

    src = _worker_layout(edge_index[0])
    dst = _worker_layout(edge_index[1])
    xp = jnp.pad(x, ((0, NPAD - N), (0, 0)))
    zeros_z = jnp.zeros((NPAD, D), jnp.float32)
    ones128 = jnp.ones((CHUNK, D), jnp.float32)

    degp = _deg_call(dst, zeros_z, ones128)
    y1, dinv = _tc1_call(xp, conv1_w, degp)
    z1 = _conv_call(y1, src, dst, zeros_z)
    y2 = _tcmid_call(z1, y1, dinv, conv1_b.reshape(1, H), conv2_w)
    z2 = _conv_call(y2, src, dst, zeros_z)
    y3 = _tcmid_call(z2, y2, dinv, conv2_b.reshape(1, H), conv3_w)
    z3 = _conv_call(y3, src, dst, zeros_z)
    out = _tcfin_call(z3, y3, dinv, conv3_b.reshape(1, H),
                      fc1_w, fc1_b.reshape(1, H),
                      fc2_w, fc2_b.reshape(1, H),
                      fc3_w, fc3_b.reshape(1, 2))
    return out[:N]
